# baseline (speedup 1.0000x reference)
import functools

import jax
import jax.numpy as jnp
from jax import lax
from jax.experimental import pallas as pl
from jax.experimental.pallas import tpu as pltpu
from jax.experimental.pallas import tpu_sc as plsc

_VOCAB = 100000
_HIDDEN = 768
_BATCH = 4
_SEQ = 2048

_NC = 2
_NS = 16
_NW = _NC * _NS
_S_PER_W = _SEQ // _NW
_ROWS_W = _S_PER_W * _BATCH
_CHUNK = 32
_NBUF = 3
_L = 16
_NG = _HIDDEN // _L


def _emb_body(src_hbm, seg_hbm, tok_tab_hbm, seg_tab_hbm, pos_hbm, out_hbm,
              idx_all, seg_all, segf_all, d_all, seg_tab_v, pos_v, buf0, buf1, buf2,
              semg0, semg1, semg2, semw0, semw1, semw2, semst):
    buf = [buf0, buf1, buf2]
    semg = [semg0, semg1, semg2]
    semw = [semw0, semw1, semw2]
    wid = lax.axis_index("s") * _NC + lax.axis_index("c")
    s_base = pl.multiple_of(wid * _S_PER_W, _S_PER_W)

    nt = _S_PER_W // _CHUNK
    n = _BATCH * nt

    def start_gather(c):
        return pltpu.async_copy(
            tok_tab_hbm.at[idx_all.at[pl.ds(c * _CHUNK, _CHUNK)]],
            buf[c % _NBUF], semg[c % _NBUF])

    icps = []
    rest = []
    for b in range(_BATCH):
        icps.append(pltpu.async_copy(
            src_hbm.at[b, pl.ds(s_base, _S_PER_W)],
            idx_all.at[pl.ds(b * _S_PER_W, _S_PER_W)], semst))
        rest.append(pltpu.async_copy(
            seg_hbm.at[b, pl.ds(s_base, _S_PER_W)],
            seg_all.at[pl.ds(b * _S_PER_W, _S_PER_W)], semw0))
    rest.append(pltpu.async_copy(seg_tab_hbm, seg_tab_v, semw1))
    rest.append(pltpu.async_copy(
        pos_hbm.at[0, pl.ds(s_base, _S_PER_W)], pos_v, semw2))

    gcp = [None] * _NBUF
    wcp = [None] * _NBUF
    icps[0].wait()
    gcp[0] = start_gather(0)
    gcp[1] = start_gather(1)
    for cp in icps[1:]:
        cp.wait()
    for cp in rest:
        cp.wait()

    for k in range(_ROWS_W // _L):
        segf_all[pl.ds(k * _L, _L)] = (
            seg_all[pl.ds(k * _L, _L)].astype(jnp.float32))

    def d_body(g, carry):
        goff = pl.multiple_of(g * _L, _L)
        d_all[pl.ds(goff, _L)] = (seg_tab_v[1, pl.ds(goff, _L)]
                                  - seg_tab_v[0, pl.ds(goff, _L)])
        return carry

    lax.fori_loop(0, _NG, d_body, 0)

    def fold_body(g, carry):
        goff = pl.multiple_of(g * _L, _L)
        s0 = seg_tab_v[0, pl.ds(goff, _L)]
        for rb in range(_S_PER_W // _L):
            for j in range(_L):
                plsc.addupdate(pos_v.at[rb * _L + j, pl.ds(goff, _L)], s0)
        return carry

    lax.fori_loop(0, _NG, fold_body, 0)

    def add_posseg(c):
        buf_ref = buf[c % _NBUF]
        pos_ref = pos_v.at[pl.ds((c % nt) * _CHUNK, _CHUNK)]
        soff = c * _CHUNK

        def g_body(g, carry):
            goff = pl.multiple_of(g * _L, _L)
            d = d_all[pl.ds(goff, _L)]

            for rb in range(_CHUNK // _L):
                segf16 = segf_all[pl.ds(soff + rb * _L, _L)]
                for j in range(_L):
                    r = rb * _L + j
                    segf = jnp.full((_L,), segf16[j])
                    p = pos_ref[r, pl.ds(goff, _L)]
                    plsc.addupdate(buf_ref.at[r, pl.ds(goff, _L)],
                                   p + segf * d)
            return carry

        lax.fori_loop(0, _NG, g_body, 0)

    for c in range(n):
        cur = c % _NBUF
        b, t = divmod(c, nt)
        gcp[cur].wait()
        add_posseg(c)
        wcp[cur] = pltpu.async_copy(
            buf[cur],
            out_hbm.at[b, pl.ds(s_base + t * _CHUNK, _CHUNK)], semw[cur])
        if c + 2 < n:
            k = (c + 2) % _NBUF
            if wcp[k] is not None:
                wcp[k].wait()
                wcp[k] = None
            gcp[k] = start_gather(c + 2)
    for w in wcp:
        if w is not None:
            w.wait()


@jax.jit
def _emb(source, segment, token_table, segment_table, pos_emb):
    mesh = plsc.VectorSubcoreMesh(
        core_axis_name="c", subcore_axis_name="s",
        num_cores=_NC, num_subcores=_NS)
    run = functools.partial(
        pl.kernel,
        out_type=jax.ShapeDtypeStruct((_BATCH, _SEQ, _HIDDEN), jnp.float32),
        mesh=mesh,
        scratch_types=[
            pltpu.VMEM((_ROWS_W,), jnp.int32),
            pltpu.VMEM((_ROWS_W,), jnp.int32),
            pltpu.VMEM((_ROWS_W,), jnp.float32),
            pltpu.VMEM((_HIDDEN,), jnp.float32),
            pltpu.VMEM((2, _HIDDEN), jnp.float32),
            pltpu.VMEM((_S_PER_W, _HIDDEN), jnp.float32),
            pltpu.VMEM((_CHUNK, _HIDDEN), jnp.float32),
            pltpu.VMEM((_CHUNK, _HIDDEN), jnp.float32),
            pltpu.VMEM((_CHUNK, _HIDDEN), jnp.float32),
            pltpu.SemaphoreType.DMA,
            pltpu.SemaphoreType.DMA,
            pltpu.SemaphoreType.DMA,
            pltpu.SemaphoreType.DMA,
            pltpu.SemaphoreType.DMA,
            pltpu.SemaphoreType.DMA,
            pltpu.SemaphoreType.DMA,
        ],
    )(_emb_body)
    return run(source, segment, token_table, segment_table, pos_emb)


def kernel(source, segment, token_table, segment_table, pos_emb):
    return _emb(source.astype(jnp.int32), segment.astype(jnp.int32),
                token_table, segment_table, pos_emb)


# speedup vs baseline: 1.0293x; 1.0293x over previous
import functools

import jax
import jax.numpy as jnp
from jax import lax
from jax.experimental import pallas as pl
from jax.experimental.pallas import tpu as pltpu
from jax.experimental.pallas import tpu_sc as plsc

_VOCAB = 100000
_HIDDEN = 768
_BATCH = 4
_SEQ = 2048

_NC = 2
_NS = 16
_NW = _NC * _NS
_S_PER_W = _SEQ // _NW
_ROWS_W = _S_PER_W * _BATCH
_CHUNK = 32
_NBUF = 3
_L = 16
_NG = _HIDDEN // _L


def _emb_body(src_hbm, seg_hbm, tok_tab_hbm, seg_tab_hbm, pos_hbm, out_hbm,
              idx_all, seg_all, segf_all, d_all, seg_tab_v, pos_v, buf0, buf1, buf2,
              semg0, semg1, semg2, semw0, semw1, semw2, semst):
    buf = [buf0, buf1, buf2]
    semg = [semg0, semg1, semg2]
    semw = [semw0, semw1, semw2]
    wid = lax.axis_index("s") * _NC + lax.axis_index("c")
    s_base = pl.multiple_of(wid * _S_PER_W, _S_PER_W)

    nt = _S_PER_W // _CHUNK
    n = _BATCH * nt

    def start_gather(c):
        return pltpu.async_copy(
            tok_tab_hbm.at[idx_all.at[pl.ds(c * _CHUNK, _CHUNK)]],
            buf[c % _NBUF], semg[c % _NBUF])

    icps = []
    rest = []
    for b in range(_BATCH):
        icps.append(pltpu.async_copy(
            src_hbm.at[b, pl.ds(s_base, _S_PER_W)],
            idx_all.at[pl.ds(b * _S_PER_W, _S_PER_W)], semst))
        rest.append(pltpu.async_copy(
            seg_hbm.at[b, pl.ds(s_base, _S_PER_W)],
            seg_all.at[pl.ds(b * _S_PER_W, _S_PER_W)], semw0))
    rest.append(pltpu.async_copy(seg_tab_hbm, seg_tab_v, semw1))
    rest.append(pltpu.async_copy(
        pos_hbm.at[0, pl.ds(s_base, _S_PER_W)], pos_v, semw2))

    gcp = [None] * _NBUF
    wcp = [None] * _NBUF
    icps[0].wait()
    gcp[0] = start_gather(0)
    gcp[1] = start_gather(1)
    for cp in icps[1:]:
        cp.wait()
    for cp in rest:
        cp.wait()

    for k in range(_ROWS_W // _L):
        segf_all[pl.ds(k * _L, _L)] = (
            seg_all[pl.ds(k * _L, _L)].astype(jnp.float32))

    def d_body(g, carry):
        goff = pl.multiple_of(g * _L, _L)
        d_all[pl.ds(goff, _L)] = (seg_tab_v[1, pl.ds(goff, _L)]
                                  - seg_tab_v[0, pl.ds(goff, _L)])
        return carry

    lax.fori_loop(0, _NG, d_body, 0)

    def add_posseg(c):
        buf_ref = buf[c % _NBUF]
        pos_ref = pos_v.at[pl.ds((c % nt) * _CHUNK, _CHUNK)]
        soff = c * _CHUNK

        def g_body(g, carry):
            goff = pl.multiple_of(g * _L, _L)
            d = d_all[pl.ds(goff, _L)]
            s0 = seg_tab_v[0, pl.ds(goff, _L)]

            for rb in range(_CHUNK // _L):
                segf16 = segf_all[pl.ds(soff + rb * _L, _L)]
                for j in range(_L):
                    r = rb * _L + j
                    segf = jnp.full((_L,), segf16[j])
                    p = pos_ref[r, pl.ds(goff, _L)]
                    plsc.addupdate(buf_ref.at[r, pl.ds(goff, _L)],
                                   p + s0 + segf * d)
            return carry

        lax.fori_loop(0, _NG, g_body, 0)

    for c in range(n):
        cur = c % _NBUF
        b, t = divmod(c, nt)
        gcp[cur].wait()
        add_posseg(c)
        wcp[cur] = pltpu.async_copy(
            buf[cur],
            out_hbm.at[b, pl.ds(s_base + t * _CHUNK, _CHUNK)], semw[cur])
        if c + 2 < n:
            k = (c + 2) % _NBUF
            if wcp[k] is not None:
                wcp[k].wait()
                wcp[k] = None
            gcp[k] = start_gather(c + 2)
    for w in wcp:
        if w is not None:
            w.wait()


@jax.jit
def _emb(source, segment, token_table, segment_table, pos_emb):
    mesh = plsc.VectorSubcoreMesh(
        core_axis_name="c", subcore_axis_name="s",
        num_cores=_NC, num_subcores=_NS)
    run = functools.partial(
        pl.kernel,
        out_type=jax.ShapeDtypeStruct((_BATCH, _SEQ, _HIDDEN), jnp.float32),
        mesh=mesh,
        scratch_types=[
            pltpu.VMEM((_ROWS_W,), jnp.int32),
            pltpu.VMEM((_ROWS_W,), jnp.int32),
            pltpu.VMEM((_ROWS_W,), jnp.float32),
            pltpu.VMEM((_HIDDEN,), jnp.float32),
            pltpu.VMEM((2, _HIDDEN), jnp.float32),
            pltpu.VMEM((_S_PER_W, _HIDDEN), jnp.float32),
            pltpu.VMEM((_CHUNK, _HIDDEN), jnp.float32),
            pltpu.VMEM((_CHUNK, _HIDDEN), jnp.float32),
            pltpu.VMEM((_CHUNK, _HIDDEN), jnp.float32),
            pltpu.SemaphoreType.DMA,
            pltpu.SemaphoreType.DMA,
            pltpu.SemaphoreType.DMA,
            pltpu.SemaphoreType.DMA,
            pltpu.SemaphoreType.DMA,
            pltpu.SemaphoreType.DMA,
            pltpu.SemaphoreType.DMA,
        ],
    )(_emb_body)
    return run(source, segment, token_table, segment_table, pos_emb)


def kernel(source, segment, token_table, segment_table, pos_emb):
    return _emb(source.astype(jnp.int32), segment.astype(jnp.int32),
                token_table, segment_table, pos_emb)


# speedup vs baseline: 1.1380x; 1.1056x over previous
import functools

import jax
import jax.numpy as jnp
from jax import lax
from jax.experimental import pallas as pl
from jax.experimental.pallas import tpu as pltpu
from jax.experimental.pallas import tpu_sc as plsc

_HIDDEN = 768
_BATCH = 4
_SEQ = 2048

_NC = 2
_NS = 16
_NW = _NC * _NS
_S_PER_W = _SEQ // _NW
_ROWS_W = _S_PER_W * _BATCH
_PCH = 16
_NT = _S_PER_W // _PCH
_NM = _NT * 2
_NBUF = 6
_L = 16
_NG = _HIDDEN // _L


def _emb_body(src_hbm, seg_hbm, tok_tab_hbm, seg_tab_hbm, pos_hbm, out_hbm,
              idx_all, seg_all, segf_all, d_all, seg_tab_v, pos0, pos1,
              buf0, buf1, buf2, buf3, buf4, buf5,
              semg0, semg1, semg2, semw0, semw1, semw2, semst, sempos):
    buf = [buf0, buf1, buf2, buf3, buf4, buf5]
    semg = [semg0, semg1, semg2]
    semw = [semw0, semw1, semw2]
    posb = [pos0, pos1]
    wid = lax.axis_index("s") * _NC + lax.axis_index("c")
    s_base = pl.multiple_of(wid * _S_PER_W, _S_PER_W)

    def macro_tb(m):
        t, pr = divmod(m, 2)
        return t, (2 * pr, 2 * pr + 1)

    def start_gathers(m):
        t, (ba, bb) = macro_tb(m)
        cps = []
        for i, b in enumerate((ba, bb)):
            k = (2 * m + i) % _NBUF
            cps.append(pltpu.async_copy(
                tok_tab_hbm.at[idx_all.at[pl.ds(b * _S_PER_W + t * _PCH,
                                                _PCH)]],
                buf[k], semg[k % 3]))
        return cps

    icps = []
    rest = []
    for b in range(_BATCH):
        icps.append(pltpu.async_copy(
            src_hbm.at[b, pl.ds(s_base, _S_PER_W)],
            idx_all.at[pl.ds(b * _S_PER_W, _S_PER_W)], semst))
        rest.append(pltpu.async_copy(
            seg_hbm.at[b, pl.ds(s_base, _S_PER_W)],
            seg_all.at[pl.ds(b * _S_PER_W, _S_PER_W)], semw0))
    rest.append(pltpu.async_copy(seg_tab_hbm, seg_tab_v, semw1))
    rest.append(pltpu.async_copy(
        pos_hbm.at[0, pl.ds(s_base, _PCH)], pos0, semw2))

    for cp in icps:
        cp.wait()
    gcps = [None] * _NM
    gcps[0] = start_gathers(0)
    gcps[1] = start_gathers(1)
    gcps[2] = start_gathers(2)
    for cp in rest:
        cp.wait()

    for k in range(_ROWS_W // _L):
        segf_all[pl.ds(k * _L, _L)] = (
            seg_all[pl.ds(k * _L, _L)].astype(jnp.float32))

    def d_body(g, carry):
        goff = pl.multiple_of(g * _L, _L)
        d_all[pl.ds(goff, _L)] = (seg_tab_v[1, pl.ds(goff, _L)]
                                  - seg_tab_v[0, pl.ds(goff, _L)])
        return carry

    lax.fori_loop(0, _NG, d_body, 0)

    def compute(m):
        t, (ba, bb) = macro_tb(m)
        bufa = buf[(2 * m) % _NBUF]
        bufb = buf[(2 * m + 1) % _NBUF]
        pos_ref = posb[t % 2]
        sa = ba * _S_PER_W + t * _PCH
        sb = bb * _S_PER_W + t * _PCH

        def g_body(g, carry):
            goff = pl.multiple_of(g * _L, _L)
            d = d_all[pl.ds(goff, _L)]
            s0 = seg_tab_v[0, pl.ds(goff, _L)]
            segfa = segf_all[pl.ds(sa, _L)]
            segfb = segf_all[pl.ds(sb, _L)]
            for j in range(_PCH):
                p = pos_ref[j, pl.ds(goff, _L)]
                tmp = p + s0
                fa = jnp.full((_L,), segfa[j])
                fb = jnp.full((_L,), segfb[j])
                plsc.addupdate(bufa.at[j, pl.ds(goff, _L)], tmp + fa * d)
                plsc.addupdate(bufb.at[j, pl.ds(goff, _L)], tmp + fb * d)
            return carry

        lax.fori_loop(0, _NG, g_body, 0)

    wcps = [None] * _NM
    pcp = [None]
    for m in range(_NM):
        t, (ba, bb) = macro_tb(m)
        if m % 2 == 0:
            if t > 0:
                pcp[0].wait()
            if t + 1 < _NT:
                pcp[0] = pltpu.async_copy(
                    pos_hbm.at[0, pl.ds(s_base + (t + 1) * _PCH, _PCH)],
                    posb[(t + 1) % 2], sempos)
        for cp in gcps[m]:
            cp.wait()
        compute(m)
        wcps[m] = [
            pltpu.async_copy(
                buf[(2 * m) % _NBUF],
                out_hbm.at[ba, pl.ds(s_base + t * _PCH, _PCH)], semw[0]),
            pltpu.async_copy(
                buf[(2 * m + 1) % _NBUF],
                out_hbm.at[bb, pl.ds(s_base + t * _PCH, _PCH)], semw[1]),
        ]
        if m >= 1 and m + 2 < _NM:
            for cp in wcps[m - 1]:
                cp.wait()
            wcps[m - 1] = None
            gcps[m + 2] = start_gathers(m + 2)
    for ws in wcps:
        if ws is not None:
            for cp in ws:
                cp.wait()


@jax.jit
def _emb(source, segment, token_table, segment_table, pos_emb):
    mesh = plsc.VectorSubcoreMesh(
        core_axis_name="c", subcore_axis_name="s",
        num_cores=_NC, num_subcores=_NS)
    run = functools.partial(
        pl.kernel,
        out_type=jax.ShapeDtypeStruct((_BATCH, _SEQ, _HIDDEN), jnp.float32),
        mesh=mesh,
        scratch_types=[
            pltpu.VMEM((_ROWS_W,), jnp.int32),
            pltpu.VMEM((_ROWS_W,), jnp.int32),
            pltpu.VMEM((_ROWS_W,), jnp.float32),
            pltpu.VMEM((_HIDDEN,), jnp.float32),
            pltpu.VMEM((2, _HIDDEN), jnp.float32),
            pltpu.VMEM((_PCH, _HIDDEN), jnp.float32),
            pltpu.VMEM((_PCH, _HIDDEN), jnp.float32),
            pltpu.VMEM((_PCH, _HIDDEN), jnp.float32),
            pltpu.VMEM((_PCH, _HIDDEN), jnp.float32),
            pltpu.VMEM((_PCH, _HIDDEN), jnp.float32),
            pltpu.VMEM((_PCH, _HIDDEN), jnp.float32),
            pltpu.VMEM((_PCH, _HIDDEN), jnp.float32),
            pltpu.VMEM((_PCH, _HIDDEN), jnp.float32),
            pltpu.SemaphoreType.DMA,
            pltpu.SemaphoreType.DMA,
            pltpu.SemaphoreType.DMA,
            pltpu.SemaphoreType.DMA,
            pltpu.SemaphoreType.DMA,
            pltpu.SemaphoreType.DMA,
            pltpu.SemaphoreType.DMA,
            pltpu.SemaphoreType.DMA,
        ],
    )(_emb_body)
    return run(source, segment, token_table, segment_table, pos_emb)


def kernel(source, segment, token_table, segment_table, pos_emb):
    return _emb(source.astype(jnp.int32), segment.astype(jnp.int32),
                token_table, segment_table, pos_emb)
